# Initial kernel scaffold; baseline (speedup 1.0000x reference)
import jax
import jax.numpy as jnp
from jax.experimental import pallas as pl


def kernel(hidden_states, W1, b1, W2, b2):
    raise NotImplementedError("write your pallas kernel here")


# speedup vs baseline: 1.5706x; 1.5706x over previous
import jax
import jax.numpy as jnp
import numpy as np
from jax.experimental import pallas as pl

B, S, D = 4, 4096, 2048
H = D // 4
K = max(1, int(0.5 * S))
TOK = 512
N_TILES = (B * S) // TOK


_ERF_C = [np.float32(c) for c in
          ("7.85386146e-05", "-0.000801019371", "0.00518832775",
           "-0.0268538129", "0.112835854", "-0.37612626", "1.12837911")]
_ERFC_PA = [np.float32(c) for c in
            ("0.0232682", "-0.138703942", "0.368742466", "-0.582473278",
             "0.621000469", "-0.494451523", "0.340488", "-0.274112701",
             "0.563825965")]
_ERFC_PB = [np.float32(c) for c in
            ("-10.477664", "12.9772", "-7.49551868", "2.92101908",
             "-1.01526523", "0.42184633", "-0.282076746", "0.564189494")]


def _gelu_exact(h):
    q = -h * np.float32("0.707106769")
    x2 = q * q
    e = x2 * _ERF_C[0]
    for c in _ERF_C[1:-1]:
        e = (e + c) * x2
    e = e + _ERF_C[-1]
    res_lt1 = 1.0 - q * e

    z = -x2
    ex = jnp.exp(z)
    absq = jnp.abs(q)
    factor = ex * (1.0 / absq)
    w = 1.0 / x2
    pa = w * _ERFC_PA[0]
    for c in _ERFC_PA[1:-1]:
        pa = (pa + c) * w
    pa = pa + _ERFC_PA[-1]
    pb = w * _ERFC_PB[0]
    for c in _ERFC_PB[1:-1]:
        pb = (pb + c) * w
    pb = pb + _ERFC_PB[-1]
    r = factor * jnp.where(absq < 2.0, pa, pb)
    r = jnp.where(z < np.float32("-88.7228394"), 0.0, r)
    r = jnp.where(q < 0.0, 2.0 - r, r)
    erfc_q = jnp.where(absq < 1.0, res_lt1, r)
    return (0.5 * h) * erfc_q


def _round_bf16_rne(a):
    u = jax.lax.bitcast_convert_type(a, jnp.uint32)
    r = u + jnp.uint32(0x7FFF) + ((u >> jnp.uint32(16)) & jnp.uint32(1))
    return jax.lax.bitcast_convert_type(r & jnp.uint32(0xFFFF0000),
                                        jnp.float32)


def _mlp_body(x_ref, w1_ref, b1_ref, w2_ref, b2_ref, out_ref):
    x = _round_bf16_rne(x_ref[...]).astype(jnp.bfloat16)
    h = jnp.dot(x, w1_ref[...], preferred_element_type=jnp.float32)
    h = h + b1_ref[...]
    g = _round_bf16_rne(_gelu_exact(h)).astype(jnp.bfloat16)
    lg = jnp.dot(g, w2_ref[...], preferred_element_type=jnp.float32)
    lg = lg[:, 0:1] + b2_ref[...]
    out_ref[...] = 1.0 / (1.0 + jnp.exp(-lg))


def _topk_body(s_ref, w_ref, m_ref):
    s = s_ref[...]
    keys = jax.lax.bitcast_convert_type(s, jnp.int32)
    idx = jax.lax.broadcasted_iota(jnp.int32, (B, S), 1)

    def vbody(_, carry):
        lo, hi = carry
        mid = jax.lax.div(lo + hi, 2)
        cnt = jnp.sum((keys >= mid).astype(jnp.int32), axis=1, keepdims=True)
        pred = cnt >= K
        return jnp.where(pred, mid, lo), jnp.where(pred, hi, mid)

    lo0 = jnp.zeros((B, 1), jnp.int32)
    hi0 = jnp.full((B, 1), 0x3F800001, jnp.int32)
    t, _ = jax.lax.fori_loop(0, 31, vbody, (lo0, hi0))

    gt = keys > t
    need = K - jnp.sum(gt.astype(jnp.int32), axis=1, keepdims=True)
    tie = keys == t

    def ibody(_, carry):
        lo, hi = carry
        mid = jax.lax.div(lo + hi, 2)
        cnt = jnp.sum((tie & (idx < mid)).astype(jnp.int32), axis=1, keepdims=True)
        pred = cnt >= need
        return jnp.where(pred, lo, mid), jnp.where(pred, mid, hi)

    _, J = jax.lax.fori_loop(0, 12, ibody,
                             (jnp.zeros((B, 1), jnp.int32),
                              jnp.full((B, 1), S, jnp.int32)))

    mask = gt | (tie & (idx < J))
    w_ref[...] = jnp.where(mask, s, 0.0)
    m_ref[...] = mask.astype(jnp.float32)


def kernel(hidden_states, W1, b1, W2, b2):
    x = hidden_states.reshape(B * S, D)
    scores = pl.pallas_call(
        _mlp_body,
        grid=(N_TILES,),
        in_specs=[
            pl.BlockSpec((TOK, D), lambda i: (i, 0)),
            pl.BlockSpec((D, H), lambda i: (0, 0)),
            pl.BlockSpec((1, H), lambda i: (0, 0)),
            pl.BlockSpec((H, 128), lambda i: (0, 0)),
            pl.BlockSpec((1, 1), lambda i: (0, 0)),
        ],
        out_specs=pl.BlockSpec((TOK, 1), lambda i: (i, 0)),
        out_shape=jax.ShapeDtypeStruct((B * S, 1), jnp.float32),
    )(x, W1.astype(jnp.bfloat16), b1.reshape(1, H),
      jnp.zeros((H, 128), jnp.bfloat16).at[:, 0].set(
          W2[:, 0].astype(jnp.bfloat16)),
      b2.reshape(1, 1))

    s = scores.reshape(B, S)
    weights, mask_f = pl.pallas_call(
        _topk_body,
        out_shape=(
            jax.ShapeDtypeStruct((B, S), jnp.float32),
            jax.ShapeDtypeStruct((B, S), jnp.float32),
        ),
    )(s)
    return weights, mask_f.astype(bool)
